# baseline (speedup 1.0000x reference)
import functools
import jax
import jax.numpy as jnp
from jax import lax
from jax.experimental import pallas as pl
from jax.experimental.pallas import tpu as pltpu
from jax.experimental.pallas import tpu_sc as plsc

E = 32
N = 640000
D = 64
NW = 32
RPW = N // NW
C = 160
G = RPW // C


def _row_compute(x0v, x1v, ov, r):
    a_r0 = x0v[r, pl.ds(0, 16)]
    a_r1 = x0v[r, pl.ds(16, 16)]
    a_i0 = x0v[r, pl.ds(32, 16)]
    a_i1 = x0v[r, pl.ds(48, 16)]
    b_r0 = x1v[r, pl.ds(0, 16)]
    b_r1 = x1v[r, pl.ds(16, 16)]
    b_i0 = x1v[r, pl.ds(32, 16)]
    b_i1 = x1v[r, pl.ds(48, 16)]
    ov[r, pl.ds(0, 16)] = a_r0 * b_r0 - a_i0 * b_i0
    ov[r, pl.ds(16, 16)] = a_r1 * b_r1 - a_i1 * b_i1
    ov[r, pl.ds(32, 16)] = a_i0 * b_r0 + a_r0 * b_i0
    ov[r, pl.ds(48, 16)] = a_i1 * b_r1 + a_r1 * b_i1


def _sc_body(in0_hbm, in1_hbm, out_hbm,
             x0a, x0b, x1a, x1b, oa, ob,
             s0a, s0b, s1a, s1b, soa, sob):
    wid = lax.axis_index("s") * 2 + lax.axis_index("c")
    base = wid * RPW

    x0 = (x0a, x0b)
    x1 = (x1a, x1b)
    o = (oa, ob)
    s0 = (s0a, s0b)
    s1 = (s1a, s1b)
    so = (soa, sob)

    def in_copies(g, b):
        rows = pl.ds(base + g * C, C)
        return (
            pltpu.make_async_copy(in0_hbm.at[rows], x0[b], s0[b]),
            pltpu.make_async_copy(in1_hbm.at[rows], x1[b], s1[b]),
        )

    def out_copy(g, b):
        rows = pl.ds(base + g * C, C)
        return pltpu.make_async_copy(o[b], out_hbm.at[rows], so[b])

    for b in (0, 1):
        for c in in_copies(b, b):
            c.start()

    def chunk(g, b, not_first):
        for c in in_copies(g, b):
            c.wait()

        @pl.when(not_first)
        def _():
            out_copy(g, b).wait()

        def row_body(r, carry):
            _row_compute(x0[b], x1[b], o[b], r)
            return carry

        lax.fori_loop(0, C, row_body, 0, unroll=4)

        out_copy(g, b).start()

        @pl.when(g + 2 < G)
        def _():
            for c in in_copies(g + 2, b):
                c.start()

    def pair(i, carry):
        chunk(2 * i, 0, i >= 1)
        chunk(2 * i + 1, 1, i >= 1)
        return carry

    lax.fori_loop(0, G // 2, pair, 0)

    if G % 2 == 1:
        chunk(G - 1, 0, jnp.bool_(True))

    for b in (0, 1):
        out_copy(0, b).wait()


def sc_kernel(in0, in1):
    scratch = (
        [pltpu.VMEM((C, D), jnp.float32) for _ in range(6)]
        + [pltpu.SemaphoreType.DMA] * 6
    )
    mesh = plsc.VectorSubcoreMesh(core_axis_name="c", subcore_axis_name="s")
    k = pl.kernel(
        _sc_body,
        out_type=jax.ShapeDtypeStruct((N, D), jnp.float32),
        mesh=mesh,
        scratch_types=scratch,
        compiler_params=pltpu.CompilerParams(use_tc_tiling_on_sc=True),
    )
    return k(in0, in1)


def kernel(in0, in1):
    return sc_kernel(in0, in1)


# speedup vs baseline: 1.0009x; 1.0009x over previous
import jax
import jax.numpy as jnp
from jax import lax
from jax.experimental import pallas as pl
from jax.experimental.pallas import tpu as pltpu
from jax.experimental.pallas import tpu_sc as plsc

N = 640000
D = 64
NW = 32
RPW = N // NW
C = 160
G = RPW // C


def _row_compute(x0v, x1v, ov, r):
    a_r0 = x0v[r, pl.ds(0, 16)]
    a_r1 = x0v[r, pl.ds(16, 16)]
    a_i0 = x0v[r, pl.ds(32, 16)]
    a_i1 = x0v[r, pl.ds(48, 16)]
    b_r0 = x1v[r, pl.ds(0, 16)]
    b_r1 = x1v[r, pl.ds(16, 16)]
    b_i0 = x1v[r, pl.ds(32, 16)]
    b_i1 = x1v[r, pl.ds(48, 16)]
    ov[r, pl.ds(0, 16)] = a_r0 * b_r0 - a_i0 * b_i0
    ov[r, pl.ds(16, 16)] = a_r1 * b_r1 - a_i1 * b_i1
    ov[r, pl.ds(32, 16)] = a_i0 * b_r0 + a_r0 * b_i0
    ov[r, pl.ds(48, 16)] = a_i1 * b_r1 + a_r1 * b_i1


def _sc_body(in0_hbm, in1_hbm, out_hbm,
             x0a, x0b, x1a, x1b, oa, ob,
             s0a, s0b, s1a, s1b, soa, sob):
    wid = lax.axis_index("s") * 2 + lax.axis_index("c")
    base = wid * RPW

    x0 = (x0a, x0b)
    x1 = (x1a, x1b)
    o = (oa, ob)
    s0 = (s0a, s0b)
    s1 = (s1a, s1b)
    so = (soa, sob)

    def in_copies(g, b):
        rows = pl.ds(base + g * C, C)
        return (
            pltpu.make_async_copy(in0_hbm.at[rows], x0[b], s0[b]),
            pltpu.make_async_copy(in1_hbm.at[rows], x1[b], s1[b]),
        )

    def out_copy(g, b):
        rows = pl.ds(base + g * C, C)
        return pltpu.make_async_copy(o[b], out_hbm.at[rows], so[b])

    for b in (0, 1):
        for c in in_copies(b, b):
            c.start()

    def chunk(g, b, not_first):
        for c in in_copies(g, b):
            c.wait()

        @pl.when(not_first)
        def _():
            out_copy(g, b).wait()

        def row_body(r, carry):
            _row_compute(x0[b], x1[b], o[b], r)
            return carry

        lax.fori_loop(0, C, row_body, 0, unroll=4)

        out_copy(g, b).start()

        @pl.when(g + 2 < G)
        def _():
            for c in in_copies(g + 2, b):
                c.start()

    def pair(i, carry):
        chunk(2 * i, 0, i >= 1)
        chunk(2 * i + 1, 1, i >= 1)
        return carry

    lax.fori_loop(0, G // 2, pair, 0)

    if G % 2 == 1:
        chunk(G - 1, 0, jnp.bool_(True))

    for b in (0, 1):
        out_copy(0, b).wait()


def kernel(in0, in1):
    scratch = (
        [pltpu.VMEM((C, D), jnp.float32) for _ in range(6)]
        + [pltpu.SemaphoreType.DMA] * 6
    )
    mesh = plsc.VectorSubcoreMesh(core_axis_name="c", subcore_axis_name="s")
    k = pl.kernel(
        _sc_body,
        out_type=jax.ShapeDtypeStruct((N, D), jnp.float32),
        mesh=mesh,
        scratch_types=scratch,
        compiler_params=pltpu.CompilerParams(use_tc_tiling_on_sc=True),
    )
    return k(in0, in1)
